# baseline (speedup 1.0000x reference)
import functools

import jax
import jax.numpy as jnp
from jax import lax
from jax.experimental import pallas as pl
from jax.experimental.pallas import tpu as pltpu
from jax.experimental.pallas import tpu_sc as plsc

_B = 4096
_S = 200
_HALF = 32
_EMBED = 64
_NC = 2
_NS = 16
_NW = _NC * _NS
_BT = _B // _NW
_L = 16
_NBUF = 2


def _issue_idx(x1t, x2t, idx1, idx2, s, col0, sem):
    cols = pl.ds(col0, _BT)
    pltpu.async_copy(x1t.at[s, cols], idx1, sem)
    pltpu.async_copy(x2t.at[s, cols], idx2, sem)


def _wait_idx(x1t, x2t, idx1, idx2, s, col0, sem):
    cols = pl.ds(col0, _BT)
    pltpu.make_async_copy(x1t.at[s, cols], idx1, sem).wait()
    pltpu.make_async_copy(x2t.at[s, cols], idx2, sem).wait()


def _issue_gather(t1, t2, idx1, idx2, r1, r2, sem):
    pltpu.async_copy(t1.at[idx1], r1, sem)
    pltpu.async_copy(t2.at[idx2], r2, sem)


def _wait_gather(t1, t2, idx1, idx2, r1, r2, sem):
    pltpu.make_async_copy(t1.at[idx1], r1, sem).wait()
    pltpu.make_async_copy(t2.at[idx2], r2, sem).wait()


def _compute(r1, r2, pe_v, ob, s, iota):
    for half, src in ((0, r1), (1, r2)):
        for g in range(_HALF // _L):
            pev = pe_v[s, pl.ds(half * _HALF + g * _L, _L)]

            @pl.loop(0, _BT // _L)
            def _chunk(c):
                rowv = iota + c * _L
                for ei in range(_L):
                    e = g * _L + ei
                    col = jnp.full((_L,), e, jnp.int32)
                    v = plsc.load_gather(src, [rowv, col]) + pev[ei]
                    ob[half * 4 + e // 8, e % 8, pl.ds(c * _L, _L)] = v


def _body(x1t_hbm, x2t_hbm, t1_hbm, t2_hbm, pe_hbm, out_hbm,
          idx1_v, idx2_v, rows1_v, rows2_v, pe_v, out_v,
          isem0, isem1, gsem0, gsem1, osem0, osem1):
    isems = (isem0, isem1)
    gsems = (gsem0, gsem1)
    osems = (osem0, osem1)
    wid = lax.axis_index("s") * _NC + lax.axis_index("c")
    col0 = wid * _BT
    pltpu.sync_copy(pe_hbm, pe_v)
    iota = lax.iota(jnp.int32, _L)

    for b in range(_NBUF):
        _issue_idx(x1t_hbm, x2t_hbm, idx1_v.at[b], idx2_v.at[b], b, col0,
                   isems[b])
    _wait_idx(x1t_hbm, x2t_hbm, idx1_v.at[0], idx2_v.at[0], 0, col0, isems[0])
    _issue_gather(t1_hbm, t2_hbm, idx1_v.at[0], idx2_v.at[0],
                  rows1_v.at[0], rows2_v.at[0], gsems[0])

    def _step(s, b, first, last):
        nb = 1 - b
        if not last:
            _wait_idx(x1t_hbm, x2t_hbm, idx1_v.at[nb], idx2_v.at[nb],
                      s + 1, col0, isems[nb])
            _issue_gather(t1_hbm, t2_hbm, idx1_v.at[nb], idx2_v.at[nb],
                          rows1_v.at[nb], rows2_v.at[nb], gsems[nb])
        _wait_gather(t1_hbm, t2_hbm, idx1_v.at[b], idx2_v.at[b],
                     rows1_v.at[b], rows2_v.at[b], gsems[b])
        if not first:
            @pl.when(s >= _NBUF)
            def _():
                pltpu.make_async_copy(out_v.at[b],
                                      out_hbm.at[s - _NBUF, :, wid],
                                      osems[b]).wait()
        _compute(rows1_v.at[b], rows2_v.at[b], pe_v, out_v.at[b], s, iota)
        pltpu.async_copy(out_v.at[b], out_hbm.at[s, :, wid], osems[b])
        if not last:
            @pl.when(s + _NBUF < _S)
            def _():
                _issue_idx(x1t_hbm, x2t_hbm, idx1_v.at[b], idx2_v.at[b],
                           s + _NBUF, col0, isems[b])

    @pl.loop(0, _S - 2, step=_NBUF)
    def _main(i):
        for b in range(_NBUF):
            _step(i + b, b, first=False, last=False)

    _step(_S - 2, 0, first=False, last=False)
    _step(_S - 1, 1, first=False, last=True)

    for b in range(_NBUF):
        s = _S - _NBUF + b
        pltpu.make_async_copy(out_v.at[b], out_hbm.at[s, :, wid],
                              osems[b]).wait()


@functools.partial(
    pl.kernel,
    out_type=jax.ShapeDtypeStruct((_S, _EMBED // 8, _B // 128, 8, 128),
                                  jnp.float32),
    mesh=plsc.VectorSubcoreMesh(core_axis_name="c", subcore_axis_name="s"),
    scratch_types=[
        pltpu.VMEM((_NBUF, _BT), jnp.int32),
        pltpu.VMEM((_NBUF, _BT), jnp.int32),
        pltpu.VMEM((_NBUF, _BT, _HALF), jnp.float32),
        pltpu.VMEM((_NBUF, _BT, _HALF), jnp.float32),
        pltpu.VMEM((_S, _EMBED), jnp.float32),
        pltpu.VMEM((_NBUF, 8, 8, 128), jnp.float32),
        pltpu.SemaphoreType.DMA,
        pltpu.SemaphoreType.DMA,
        pltpu.SemaphoreType.DMA,
        pltpu.SemaphoreType.DMA,
        pltpu.SemaphoreType.DMA,
        pltpu.SemaphoreType.DMA,
    ],
    compiler_params=pltpu.CompilerParams(use_tc_tiling_on_sc=False,
                                         needs_layout_passes=False,
                                         disable_bounds_checks=True),
)
def _emb_kernel(*refs):
    _body(*refs)


def kernel(x1, x2, table1, table2, pe):
    x1t = x1.astype(jnp.int32).T
    x2t = x2.astype(jnp.int32).T
    pe2d = pe.reshape(_S, _EMBED)
    y5 = _emb_kernel(x1t, x2t, table1, table2, pe2d)
    return jnp.transpose(y5, (2, 4, 0, 1, 3)).reshape(_B, _S, _EMBED)


# speedup vs baseline: 1.4934x; 1.4934x over previous
import functools

import jax
import jax.numpy as jnp
from jax import lax
from jax.experimental import pallas as pl
from jax.experimental.pallas import tpu as pltpu
from jax.experimental.pallas import tpu_sc as plsc

_B = 4096
_S = 200
_HALF = 32
_EMBED = 64
_NC = 2
_NS = 16
_NW = _NC * _NS
_BT = _B // _NW
_L = 16
_NBUF = 2


def _issue_idx(x1t, x2t, idx1, idx2, s, col0, sem):
    cols = pl.ds(col0, _BT)
    pltpu.async_copy(x1t.at[s, cols], idx1, sem)
    pltpu.async_copy(x2t.at[s, cols], idx2, sem)


def _wait_idx(x1t, x2t, idx1, idx2, s, col0, sem):
    cols = pl.ds(col0, _BT)
    pltpu.make_async_copy(x1t.at[s, cols], idx1, sem).wait()
    pltpu.make_async_copy(x2t.at[s, cols], idx2, sem).wait()


def _issue_gather(t1, t2, idx1, idx2, r1, r2, sem):
    pltpu.async_copy(t1.at[idx1], r1, sem)
    pltpu.async_copy(t2.at[idx2], r2, sem)


def _wait_gather(t1, t2, idx1, idx2, r1, r2, sem):
    pltpu.make_async_copy(t1.at[idx1], r1, sem).wait()
    pltpu.make_async_copy(t2.at[idx2], r2, sem).wait()


def _compute(r1, r2, pe_v, ob, s, iota):
    for half, src in ((0, r1), (1, r2)):
        for g in range(_HALF // _L):
            pev = pe_v[s, pl.ds(half * _HALF + g * _L, _L)]

            @plsc.parallel_loop(0, _BT // _L, unroll=2)
            def _chunk(c):
                rowv = iota + c * _L
                for ei in range(_L):
                    e = g * _L + ei
                    col = jnp.full((_L,), e, jnp.int32)
                    v = plsc.load_gather(src, [rowv, col]) + pev[ei]
                    ob[half * 4 + e // 8, e % 8, pl.ds(c * _L, _L)] = v


def _body(x1t_hbm, x2t_hbm, t1_hbm, t2_hbm, pe_hbm, out_hbm,
          idx1_v, idx2_v, rows1_v, rows2_v, pe_v, out_v,
          isem0, isem1, gsem0, gsem1, osem0, osem1):
    isems = (isem0, isem1)
    gsems = (gsem0, gsem1)
    osems = (osem0, osem1)
    wid = lax.axis_index("s") * _NC + lax.axis_index("c")
    col0 = wid * _BT
    pltpu.sync_copy(pe_hbm, pe_v)
    iota = lax.iota(jnp.int32, _L)

    for b in range(_NBUF):
        _issue_idx(x1t_hbm, x2t_hbm, idx1_v.at[b], idx2_v.at[b], b, col0,
                   isems[b])
    _wait_idx(x1t_hbm, x2t_hbm, idx1_v.at[0], idx2_v.at[0], 0, col0, isems[0])
    _issue_gather(t1_hbm, t2_hbm, idx1_v.at[0], idx2_v.at[0],
                  rows1_v.at[0], rows2_v.at[0], gsems[0])

    def _step(s, b, first, last):
        nb = 1 - b
        if not last:
            _wait_idx(x1t_hbm, x2t_hbm, idx1_v.at[nb], idx2_v.at[nb],
                      s + 1, col0, isems[nb])
            _issue_gather(t1_hbm, t2_hbm, idx1_v.at[nb], idx2_v.at[nb],
                          rows1_v.at[nb], rows2_v.at[nb], gsems[nb])
        _wait_gather(t1_hbm, t2_hbm, idx1_v.at[b], idx2_v.at[b],
                     rows1_v.at[b], rows2_v.at[b], gsems[b])
        if not first:
            @pl.when(s >= _NBUF)
            def _():
                pltpu.make_async_copy(out_v.at[b],
                                      out_hbm.at[s - _NBUF, :, wid],
                                      osems[b]).wait()
        _compute(rows1_v.at[b], rows2_v.at[b], pe_v, out_v.at[b], s, iota)
        pltpu.async_copy(out_v.at[b], out_hbm.at[s, :, wid], osems[b])
        if not last:
            @pl.when(s + _NBUF < _S)
            def _():
                _issue_idx(x1t_hbm, x2t_hbm, idx1_v.at[b], idx2_v.at[b],
                           s + _NBUF, col0, isems[b])

    @pl.loop(0, _S - 2, step=_NBUF)
    def _main(i):
        for b in range(_NBUF):
            _step(i + b, b, first=False, last=False)

    _step(_S - 2, 0, first=False, last=False)
    _step(_S - 1, 1, first=False, last=True)

    for b in range(_NBUF):
        s = _S - _NBUF + b
        pltpu.make_async_copy(out_v.at[b], out_hbm.at[s, :, wid],
                              osems[b]).wait()


@functools.partial(
    pl.kernel,
    out_type=jax.ShapeDtypeStruct((_S, _EMBED // 8, _B // 128, 8, 128),
                                  jnp.float32),
    mesh=plsc.VectorSubcoreMesh(core_axis_name="c", subcore_axis_name="s"),
    scratch_types=[
        pltpu.VMEM((_NBUF, _BT), jnp.int32),
        pltpu.VMEM((_NBUF, _BT), jnp.int32),
        pltpu.VMEM((_NBUF, _BT, _HALF), jnp.float32),
        pltpu.VMEM((_NBUF, _BT, _HALF), jnp.float32),
        pltpu.VMEM((_S, _EMBED), jnp.float32),
        pltpu.VMEM((_NBUF, 8, 8, 128), jnp.float32),
        pltpu.SemaphoreType.DMA,
        pltpu.SemaphoreType.DMA,
        pltpu.SemaphoreType.DMA,
        pltpu.SemaphoreType.DMA,
        pltpu.SemaphoreType.DMA,
        pltpu.SemaphoreType.DMA,
    ],
    compiler_params=pltpu.CompilerParams(use_tc_tiling_on_sc=False,
                                         needs_layout_passes=False,
                                         disable_bounds_checks=True),
)
def _emb_kernel(*refs):
    _body(*refs)


def kernel(x1, x2, table1, table2, pe):
    x1t = x1.astype(jnp.int32).T
    x2t = x2.astype(jnp.int32).T
    pe2d = pe.reshape(_S, _EMBED)
    y5 = _emb_kernel(x1t, x2t, table1, table2, pe2d)
    return jnp.transpose(y5, (2, 4, 0, 1, 3)).reshape(_B, _S, _EMBED)


# speedup vs baseline: 4.1970x; 2.8104x over previous
import functools

import jax
import jax.numpy as jnp
from jax import lax
from jax.experimental import pallas as pl
from jax.experimental.pallas import tpu as pltpu
from jax.experimental.pallas import tpu_sc as plsc

_B = 4096
_S = 200
_HALF = 32
_EMBED = 64
_NC = 2
_NS = 16
_NW = _NC * _NS
_BT = _B // _NW
_L = 16
_NBUF = 2
_OPITCH = 129


def _issue_idx(x1t, x2t, idx1, idx2, s, col0, sem):
    cols = pl.ds(col0, _BT)
    pltpu.async_copy(x1t.at[s, cols], idx1, sem)
    pltpu.async_copy(x2t.at[s, cols], idx2, sem)


def _wait_idx(x1t, x2t, idx1, idx2, s, col0, sem):
    cols = pl.ds(col0, _BT)
    pltpu.make_async_copy(x1t.at[s, cols], idx1, sem).wait()
    pltpu.make_async_copy(x2t.at[s, cols], idx2, sem).wait()


def _issue_gather(t1, t2, idx1, idx2, r1, r2, sem):
    pltpu.async_copy(t1.at[idx1], r1, sem)
    pltpu.async_copy(t2.at[idx2], r2, sem)


def _wait_gather(t1, t2, idx1, idx2, r1, r2, sem):
    pltpu.make_async_copy(t1.at[idx1], r1, sem).wait()
    pltpu.make_async_copy(t2.at[idx2], r2, sem).wait()


def _compute(r1, r2, pe_v, ob, s, iota):
    pev = []
    etv = []
    eiv = []
    for g in range(_EMBED // _L):
        pev.append(pe_v[s, pl.ds(g * _L, _L)])
        ev = iota + g * _L
        etv.append(ev >> 3)
        eiv.append(ev & 7)

    @plsc.parallel_loop(0, _BT, unroll=4)
    def _row(b):
        bv = jnp.full((_L,), b, jnp.int32)
        for g in range(_EMBED // _L):
            src = r1 if g < 2 else r2
            v = src[b, pl.ds((g % 2) * _L, _L)] + pev[g]
            plsc.store_scatter(ob, [etv[g], eiv[g], bv], v)


def _body(x1t_hbm, x2t_hbm, t1_hbm, t2_hbm, pe_hbm, out_hbm,
          idx1_v, idx2_v, rows1_v, rows2_v, pe_v, out_v,
          isem0, isem1, gsem0, gsem1, osem0, osem1):
    isems = (isem0, isem1)
    gsems = (gsem0, gsem1)
    osems = (osem0, osem1)
    wid = lax.axis_index("s") * _NC + lax.axis_index("c")
    col0 = wid * _BT
    pltpu.sync_copy(pe_hbm, pe_v)
    iota = lax.iota(jnp.int32, _L)

    for b in range(_NBUF):
        _issue_idx(x1t_hbm, x2t_hbm, idx1_v.at[b], idx2_v.at[b], b, col0,
                   isems[b])
    _wait_idx(x1t_hbm, x2t_hbm, idx1_v.at[0], idx2_v.at[0], 0, col0, isems[0])
    _issue_gather(t1_hbm, t2_hbm, idx1_v.at[0], idx2_v.at[0],
                  rows1_v.at[0], rows2_v.at[0], gsems[0])

    def _step(s, b, first, last):
        nb = 1 - b
        if not last:
            _wait_idx(x1t_hbm, x2t_hbm, idx1_v.at[nb], idx2_v.at[nb],
                      s + 1, col0, isems[nb])
            _issue_gather(t1_hbm, t2_hbm, idx1_v.at[nb], idx2_v.at[nb],
                          rows1_v.at[nb], rows2_v.at[nb], gsems[nb])
        _wait_gather(t1_hbm, t2_hbm, idx1_v.at[b], idx2_v.at[b],
                     rows1_v.at[b], rows2_v.at[b], gsems[b])
        if not first:
            @pl.when(s >= _NBUF)
            def _():
                pltpu.make_async_copy(
                    out_v.at[b, :, :, pl.ds(0, 128)],
                    out_hbm.at[s - _NBUF, :, wid], osems[b]).wait()
        _compute(rows1_v.at[b], rows2_v.at[b], pe_v, out_v.at[b], s, iota)
        pltpu.async_copy(out_v.at[b, :, :, pl.ds(0, 128)],
                         out_hbm.at[s, :, wid], osems[b])
        if not last:
            @pl.when(s + _NBUF < _S)
            def _():
                _issue_idx(x1t_hbm, x2t_hbm, idx1_v.at[b], idx2_v.at[b],
                           s + _NBUF, col0, isems[b])

    @pl.loop(0, _S - 2, step=_NBUF)
    def _main(i):
        for b in range(_NBUF):
            _step(i + b, b, first=False, last=False)

    _step(_S - 2, 0, first=False, last=False)
    _step(_S - 1, 1, first=False, last=True)

    for b in range(_NBUF):
        s = _S - _NBUF + b
        pltpu.make_async_copy(out_v.at[b, :, :, pl.ds(0, 128)],
                              out_hbm.at[s, :, wid], osems[b]).wait()


@functools.partial(
    pl.kernel,
    out_type=jax.ShapeDtypeStruct((_S, _EMBED // 8, _B // 128, 8, 128),
                                  jnp.float32),
    mesh=plsc.VectorSubcoreMesh(core_axis_name="c", subcore_axis_name="s"),
    scratch_types=[
        pltpu.VMEM((_NBUF, _BT), jnp.int32),
        pltpu.VMEM((_NBUF, _BT), jnp.int32),
        pltpu.VMEM((_NBUF, _BT, _HALF), jnp.float32),
        pltpu.VMEM((_NBUF, _BT, _HALF), jnp.float32),
        pltpu.VMEM((_S, _EMBED), jnp.float32),
        pltpu.VMEM((_NBUF, 8, 8, _OPITCH), jnp.float32),
        pltpu.SemaphoreType.DMA,
        pltpu.SemaphoreType.DMA,
        pltpu.SemaphoreType.DMA,
        pltpu.SemaphoreType.DMA,
        pltpu.SemaphoreType.DMA,
        pltpu.SemaphoreType.DMA,
    ],
    compiler_params=pltpu.CompilerParams(use_tc_tiling_on_sc=False,
                                         needs_layout_passes=False,
                                         disable_bounds_checks=True),
)
def _emb_kernel(*refs):
    _body(*refs)


def kernel(x1, x2, table1, table2, pe):
    x1t = x1.astype(jnp.int32).T
    x2t = x2.astype(jnp.int32).T
    pe2d = pe.reshape(_S, _EMBED)
    y5 = _emb_kernel(x1t, x2t, table1, table2, pe2d)
    return jnp.transpose(y5, (2, 4, 0, 1, 3)).reshape(_B, _S, _EMBED)


# speedup vs baseline: 4.4926x; 1.0704x over previous
import functools

import jax
import jax.numpy as jnp
from jax import lax
from jax.experimental import pallas as pl
from jax.experimental.pallas import tpu as pltpu
from jax.experimental.pallas import tpu_sc as plsc

_B = 4096
_S = 200
_HALF = 32
_EMBED = 64
_NC = 2
_NS = 16
_NW = _NC * _NS
_BT = _B // _NW
_L = 16
_NBUF = 3
_OPITCH = 129


def _compute(r1, r2, pe_v, ob, s, iota):
    pev = []
    etv = []
    eiv = []
    for g in range(_EMBED // _L):
        pev.append(pe_v[s, pl.ds(g * _L, _L)])
        ev = iota + g * _L
        etv.append(ev >> 3)
        eiv.append(ev & 7)

    @plsc.parallel_loop(0, _BT, unroll=4)
    def _row(b):
        bv = jnp.full((_L,), b, jnp.int32)
        for g in range(_EMBED // _L):
            src = r1 if g < 2 else r2
            v = src[b, pl.ds((g % 2) * _L, _L)] + pev[g]
            plsc.store_scatter(ob, [etv[g], eiv[g], bv], v)


def _body(x1t_hbm, x2t_hbm, t1_hbm, t2_hbm, pe_hbm, out_hbm,
          idx1_v, idx2_v, rows1_v, rows2_v, pe_v, out_v,
          gsem0, gsem1, gsem2, osem0, osem1, osem2):
    gsems = (gsem0, gsem1, gsem2)
    osems = (osem0, osem1, osem2)
    wid = lax.axis_index("s") * _NC + lax.axis_index("c")
    cols = pl.ds(wid * _BT, _BT)
    pltpu.sync_copy(pe_hbm, pe_v)
    pltpu.sync_copy(x1t_hbm.at[:, cols], idx1_v)
    pltpu.sync_copy(x2t_hbm.at[:, cols], idx2_v)
    iota = lax.iota(jnp.int32, _L)

    def _gather(s, b):
        pltpu.async_copy(t1_hbm.at[idx1_v.at[s]], rows1_v.at[b], gsems[b])
        pltpu.async_copy(t2_hbm.at[idx2_v.at[s]], rows2_v.at[b], gsems[b])

    def _gwait(s, b):
        pltpu.make_async_copy(t1_hbm.at[idx1_v.at[s]], rows1_v.at[b],
                              gsems[b]).wait()
        pltpu.make_async_copy(t2_hbm.at[idx2_v.at[s]], rows2_v.at[b],
                              gsems[b]).wait()

    def _owait(s, b):
        pltpu.make_async_copy(out_v.at[b, :, :, pl.ds(0, 128)],
                              out_hbm.at[s, :, wid], osems[b]).wait()

    _gather(0, 0)
    _gather(1, 1)

    def _step(s, b, prefetch):
        if prefetch:
            _gather(s + 2, (b + 2) % _NBUF)
        _gwait(s, b)

        @pl.when(s >= _NBUF)
        def _():
            _owait(s - _NBUF, b)

        _compute(rows1_v.at[b], rows2_v.at[b], pe_v, out_v.at[b], s, iota)
        pltpu.async_copy(out_v.at[b, :, :, pl.ds(0, 128)],
                         out_hbm.at[s, :, wid], osems[b])

    @pl.loop(0, _S - 2, step=_NBUF)
    def _main(i):
        for b in range(_NBUF):
            _step(i + b, b, prefetch=True)

    _step(_S - 2, (_S - 2) % _NBUF, prefetch=False)
    _step(_S - 1, (_S - 1) % _NBUF, prefetch=False)

    for s in range(_S - _NBUF, _S):
        _owait(s, s % _NBUF)


@functools.partial(
    pl.kernel,
    out_type=jax.ShapeDtypeStruct((_S, _EMBED // 8, _B // 128, 8, 128),
                                  jnp.float32),
    mesh=plsc.VectorSubcoreMesh(core_axis_name="c", subcore_axis_name="s"),
    scratch_types=[
        pltpu.VMEM((_S, _BT), jnp.int32),
        pltpu.VMEM((_S, _BT), jnp.int32),
        pltpu.VMEM((_NBUF, _BT, _HALF), jnp.float32),
        pltpu.VMEM((_NBUF, _BT, _HALF), jnp.float32),
        pltpu.VMEM((_S, _EMBED), jnp.float32),
        pltpu.VMEM((_NBUF, 8, 8, _OPITCH), jnp.float32),
        pltpu.SemaphoreType.DMA,
        pltpu.SemaphoreType.DMA,
        pltpu.SemaphoreType.DMA,
        pltpu.SemaphoreType.DMA,
        pltpu.SemaphoreType.DMA,
        pltpu.SemaphoreType.DMA,
    ],
    compiler_params=pltpu.CompilerParams(use_tc_tiling_on_sc=False,
                                         needs_layout_passes=False,
                                         disable_bounds_checks=True),
)
def _emb_kernel(*refs):
    _body(*refs)


def kernel(x1, x2, table1, table2, pe):
    x1t = x1.astype(jnp.int32).T
    x2t = x2.astype(jnp.int32).T
    pe2d = pe.reshape(_S, _EMBED)
    y5 = _emb_kernel(x1t, x2t, table1, table2, pe2d)
    return jnp.transpose(y5, (2, 4, 0, 1, 3)).reshape(_B, _S, _EMBED)
